# Initial kernel scaffold; baseline (speedup 1.0000x reference)
import jax
import jax.numpy as jnp
from jax.experimental import pallas as pl


def kernel(x, edge_index, W1, b1, W2, b2, temp):
    raise NotImplementedError("write your pallas kernel here")


# speedup vs baseline: 10.7903x; 10.7903x over previous
import functools

import jax
import jax.numpy as jnp
from jax import lax
from jax.experimental import pallas as pl
from jax.experimental.pallas import tpu as pltpu
from jax.experimental.pallas import tpu_sc as plsc

N = 10000
E = 320000
NFEAT = 128
NHID = 256
NCLASS = 64
K = 10

NC = 2
NT = 16
F = NCLASS // NC
CH = 128

NP = -(-N // (NT * 2 * 160)) * (NT * 2 * 160)
RN = NP // NT
HB = RN // 2
EP = -(-E // (NC * NT * CH)) * (NC * NT * CH)
NJ3 = EP // (NT * CH)
NJ1 = EP // (NC * NT * CH)

RB = 1024


def _k1_body(cols_hbm, deg01_hbm, deg_sh, cols_b, ones_v, zrow):
    c = lax.axis_index("c")
    s = lax.axis_index("s")
    w = c * NT + s
    r0 = s * RN

    z16 = jnp.zeros((16,), jnp.float32)
    o16 = jnp.ones((16,), jnp.float32)

    def _z(i, carry):
        zrow[pl.ds(i * 16, 16)] = z16
        return carry
    lax.fori_loop(0, RN // 16, _z, 0)

    def _o(i, carry):
        ones_v[pl.ds(i * 16, 16)] = o16
        return carry
    lax.fori_loop(0, CH // 16, _o, 0)

    pltpu.sync_copy(zrow, deg_sh.at[pl.ds(r0, RN)])
    plsc.subcore_barrier()

    def _edge(j, carry):
        pltpu.sync_copy(cols_hbm.at[w, j], cols_b)
        pltpu.sync_copy(ones_v, deg_sh.at[cols_b], add=True)
        return carry
    lax.fori_loop(0, NJ1, _edge, 0)
    plsc.subcore_barrier()

    pltpu.sync_copy(deg_sh.at[pl.ds(r0, RN)], zrow)
    pltpu.sync_copy(zrow, deg01_hbm.at[c, pl.ds(r0, RN)])


_k1 = functools.partial(
    pl.kernel,
    out_type=jax.ShapeDtypeStruct((NC, NP), jnp.float32),
    mesh=plsc.VectorSubcoreMesh(core_axis_name="c", subcore_axis_name="s"),
    compiler_params=pltpu.CompilerParams(use_tc_tiling_on_sc=False),
    scratch_types=[
        pltpu.VMEM_SHARED((NP,), jnp.float32),
        pltpu.VMEM((CH,), jnp.int32),
        pltpu.VMEM((CH,), jnp.float32),
        pltpu.VMEM((RN,), jnp.float32),
    ],
)(_k1_body)


def _k2_body(x_ref, w1_ref, b1_ref, w2_ref, b2_ref, deg_ref,
             u0_ref, invdeg_ref, sqd_ref):
    i = pl.program_id(0)
    xb = x_ref[...]
    h1 = lax.dot_general(xb, w1_ref[...], (((1,), (1,)), ((), ())),
                         preferred_element_type=jnp.float32) + b1_ref[...]
    h1 = jnp.maximum(h1, 0.0)
    h0 = lax.dot_general(h1, w2_ref[...], (((1,), (1,)), ((), ())),
                         preferred_element_type=jnp.float32) + b2_ref[...]
    degb = 1.0 + deg_ref[:, 0:1] + deg_ref[:, 1:2]
    rows = i * RB + lax.broadcasted_iota(jnp.int32, (RB, 1), 0)
    u0 = jnp.where(rows < N, h0 * lax.rsqrt(degb), 0.0)
    u0_ref[0] = u0[:, :F]
    u0_ref[1] = u0[:, F:]
    invdeg_ref[...] = jnp.broadcast_to(1.0 / degb, (RB, F))
    sqd_ref[...] = jnp.sqrt(degb)


_k2 = pl.pallas_call(
    _k2_body,
    grid=(NP // RB,),
    in_specs=[
        pl.BlockSpec((RB, NFEAT), lambda i: (i, 0)),
        pl.BlockSpec((NHID, NFEAT), lambda i: (0, 0)),
        pl.BlockSpec((1, NHID), lambda i: (0, 0)),
        pl.BlockSpec((NCLASS, NHID), lambda i: (0, 0)),
        pl.BlockSpec((1, NCLASS), lambda i: (0, 0)),
        pl.BlockSpec((RB, 2), lambda i: (i, 0)),
    ],
    out_specs=[
        pl.BlockSpec((NC, RB, F), lambda i: (0, i, 0)),
        pl.BlockSpec((RB, F), lambda i: (i, 0)),
        pl.BlockSpec((RB, 1), lambda i: (i, 0)),
    ],
    out_shape=[
        jax.ShapeDtypeStruct((NC, NP, F), jnp.float32),
        jax.ShapeDtypeStruct((NP, F), jnp.float32),
        jax.ShapeDtypeStruct((NP, 1), jnp.float32),
    ],
)


def _k3_body(u0_hbm, rows_hbm, cols_hbm, invdeg_hbm, temp_hbm, acc_hbm,
             u_sh, s_sh, rows_b, cols_b, msg, acc_t, w_t,
             stg_a, stg_b, temp_v):
    c = lax.axis_index("c")
    s = lax.axis_index("s")
    r0 = s * RN

    z16 = jnp.zeros((16,), jnp.float32)

    pltpu.sync_copy(temp_hbm, temp_v)
    pltpu.sync_copy(invdeg_hbm.at[pl.ds(r0, RN)], w_t)

    t0 = temp_v[0, pl.ds(0, 16)]
    for blk in range(RN // HB):
        rb = r0 + blk * HB
        pltpu.sync_copy(u0_hbm.at[c, pl.ds(rb, HB)], stg_a)

        def _a(i, carry, blk=blk):
            for h in range(2):
                sl = pl.ds(h * 16, 16)
                acc_t[blk * HB + i, sl] = t0 * stg_a[i, sl]
                stg_b[i, sl] = z16
            return carry
        lax.fori_loop(0, HB, _a, 0)
        pltpu.sync_copy(stg_a, u_sh.at[pl.ds(rb, HB)])
        pltpu.sync_copy(stg_b, s_sh.at[pl.ds(rb, HB)])
    plsc.subcore_barrier()

    def _hop(k, carry):
        def _edge(j, ecarry):
            pltpu.sync_copy(rows_hbm.at[s, j], rows_b)
            pltpu.sync_copy(cols_hbm.at[s, j], cols_b)
            pltpu.sync_copy(u_sh.at[rows_b], msg)
            pltpu.sync_copy(msg, s_sh.at[cols_b], add=True)
            return ecarry
        lax.fori_loop(0, NJ3, _edge, 0)
        plsc.subcore_barrier()

        tk = temp_v[k, pl.ds(0, 16)]
        for blk in range(RN // HB):
            rb = r0 + blk * HB
            pltpu.sync_copy(s_sh.at[pl.ds(rb, HB)], stg_a)
            pltpu.sync_copy(u_sh.at[pl.ds(rb, HB)], stg_b)

            def _n(i, ncarry, blk=blk):
                for h in range(2):
                    sl = pl.ds(h * 16, 16)
                    un = (stg_a[i, sl] + stg_b[i, sl]) * w_t[blk * HB + i, sl]
                    stg_b[i, sl] = un
                    stg_a[i, sl] = z16
                    acc_t[blk * HB + i, sl] = acc_t[blk * HB + i, sl] + tk * un
                return ncarry
            lax.fori_loop(0, HB, _n, 0)
            pltpu.sync_copy(stg_b, u_sh.at[pl.ds(rb, HB)])
            pltpu.sync_copy(stg_a, s_sh.at[pl.ds(rb, HB)])
        plsc.subcore_barrier()
        return carry
    lax.fori_loop(1, K + 1, _hop, 0)

    pltpu.sync_copy(acc_t, acc_hbm.at[c, pl.ds(r0, RN)])


_k3 = functools.partial(
    pl.kernel,
    out_type=jax.ShapeDtypeStruct((NC, NP, F), jnp.float32),
    mesh=plsc.VectorSubcoreMesh(core_axis_name="c", subcore_axis_name="s"),
    compiler_params=pltpu.CompilerParams(use_tc_tiling_on_sc=False),
    scratch_types=[
        pltpu.VMEM_SHARED((NP, F), jnp.float32),
        pltpu.VMEM_SHARED((NP, F), jnp.float32),
        pltpu.VMEM((CH,), jnp.int32),
        pltpu.VMEM((CH,), jnp.int32),
        pltpu.VMEM((CH, F), jnp.float32),
        pltpu.VMEM((RN, F), jnp.float32),
        pltpu.VMEM((RN, F), jnp.float32),
        pltpu.VMEM((HB, F), jnp.float32),
        pltpu.VMEM((HB, F), jnp.float32),
        pltpu.VMEM((16, 16), jnp.float32),
    ],
)(_k3_body)


def _k4_body(acc_ref, sqd_ref, out_ref):
    v = jnp.concatenate([acc_ref[0], acc_ref[1]], axis=1) * sqd_ref[...]
    m = jnp.max(v, axis=1, keepdims=True)
    e = jnp.exp(v - m)
    lse = jnp.log(jnp.sum(e, axis=1, keepdims=True))
    out_ref[...] = v - m - lse


_k4 = pl.pallas_call(
    _k4_body,
    grid=(NP // RB,),
    in_specs=[
        pl.BlockSpec((NC, RB, F), lambda i: (0, i, 0)),
        pl.BlockSpec((RB, 1), lambda i: (i, 0)),
    ],
    out_specs=pl.BlockSpec((RB, NCLASS), lambda i: (i, 0)),
    out_shape=jax.ShapeDtypeStruct((NP, NCLASS), jnp.float32),
)


def kernel(x, edge_index, W1, b1, W2, b2, temp):
    xp = jnp.pad(x, ((0, NP - N), (0, 0)))
    rows = edge_index[0]
    cols = edge_index[1]
    npad = EP - E
    padn = (N + (jnp.arange(npad, dtype=jnp.int32) % (NP - N))).astype(jnp.int32)
    rows3 = jnp.concatenate([rows, padn]).reshape(NT, NJ3, CH)
    cols_flat = jnp.concatenate([cols, padn])
    cols3 = cols_flat.reshape(NT, NJ3, CH)
    cols1 = cols_flat.reshape(NC * NT, NJ1, CH)
    temp_pad = jnp.pad(temp.astype(jnp.float32), (0, 16 - (K + 1)))
    temp_exp = jnp.tile(temp_pad[:, None], (1, 16))

    deg01 = _k1(cols1)
    degT = deg01.T
    u0, invdeg, sqd = _k2(xp, W1, b1.reshape(1, NHID), W2, b2.reshape(1, NCLASS), degT)
    acc = _k3(u0, rows3, cols3, invdeg, temp_exp)
    outp = _k4(acc, sqd)
    return outp[:N]
